# Initial kernel scaffold; baseline (speedup 1.0000x reference)
import jax
import jax.numpy as jnp
from jax.experimental import pallas as pl


def kernel(input_ids, tids, embed_table, prompt_table, task_table, W1, b1, W2, b2):
    raise NotImplementedError("write your pallas kernel here")


# speedup vs baseline: 2.0681x; 2.0681x over previous
import functools

import jax
import jax.numpy as jnp
from jax import lax
from jax.experimental import pallas as pl
from jax.experimental.pallas import tpu as pltpu
from jax.experimental.pallas import tpu_sc as plsc


def _mlp_body(p_ref, w1_ref, b1_ref, w2_ref, b2_ref, o_ref):
    h = jnp.dot(p_ref[...], w1_ref[...], preferred_element_type=jnp.float32,
                precision=lax.Precision.HIGHEST) + b1_ref[...]
    h = jnp.maximum(h, 0.0)
    o_ref[...] = jnp.dot(h, w2_ref[...], preferred_element_type=jnp.float32,
                         precision=lax.Precision.HIGHEST) + b2_ref[...]


@functools.lru_cache(maxsize=None)
def _make_sc_gather(n_rows, vocab, n_prompt, d, seq_len):
    info = plsc.get_sparse_core_info()
    nc, ns, L = info.num_cores, info.num_subcores, info.num_lanes
    nw = nc * ns
    rpw = n_rows // nw
    CH = 64
    n_ch = rpw // CH
    vec_per_ch = CH // L

    mesh = plsc.VectorSubcoreMesh(core_axis_name="c", subcore_axis_name="s")

    @functools.partial(
        pl.kernel, mesh=mesh,
        out_type=jax.ShapeDtypeStruct((n_rows, d), jnp.float32),
        scratch_types=[
            pltpu.VMEM((rpw,), jnp.int32),
            pltpu.VMEM((rpw,), jnp.int32),
            pltpu.VMEM((CH, d), jnp.float32),
            pltpu.SemaphoreType.DMA,
        ],
    )
    def sc_fn(ids_hbm, table_hbm, enc_hbm, out_hbm, ids_v, cln_v, buf_v, sem):
        wid = lax.axis_index("s") * nc + lax.axis_index("c")
        base = wid * rpw
        enc_base = (wid * rpw // seq_len) * n_prompt
        pltpu.sync_copy(ids_hbm.at[pl.ds(base, rpw)], ids_v)

        def build(v, carry):
            off = pl.multiple_of(v * L, L)
            ids16 = ids_v[pl.ds(off, L)]
            cln_v[pl.ds(off, L)] = jnp.where(ids16 >= vocab, 0, ids16)
            return carry
        lax.fori_loop(0, rpw // L, build, 0)

        def do_chunk(g, carry):
            cbase = pl.multiple_of(g * CH, CH)
            pltpu.async_copy(
                table_hbm.at[cln_v.at[pl.ds(cbase, CH)]], buf_v, sem).wait()

            def do_vec(vv, carry2):
                off = pl.multiple_of(cbase + vv * L, L)
                ids16 = ids_v[pl.ds(off, L)]
                for lidx in range(L):
                    idl = ids16[lidx]

                    @pl.when(idl >= vocab)
                    def _(idl=idl, lidx=lidx, vv=vv):
                        erow = enc_base + jnp.minimum(
                            idl - vocab, n_prompt - 1)
                        pltpu.sync_copy(enc_hbm.at[erow],
                                        buf_v.at[vv * L + lidx])
                return carry2
            lax.fori_loop(0, vec_per_ch, do_vec, 0)

            pltpu.sync_copy(buf_v, out_hbm.at[pl.ds(base + cbase, CH)])
            return carry
        lax.fori_loop(0, n_ch, do_chunk, 0)

    return sc_fn


def kernel(input_ids, tids, embed_table, prompt_table, task_table, W1, b1, W2, b2):
    B, S = input_ids.shape
    vocab, d = embed_table.shape
    n_prompt = prompt_table.shape[0]

    task_rows = jnp.take(task_table, tids, axis=0)
    p4 = (prompt_table[None, :, :] + task_rows[:, None, :]).reshape(
        B * n_prompt, d)

    enc = pl.pallas_call(
        _mlp_body,
        out_shape=jax.ShapeDtypeStruct((B * n_prompt, d), jnp.float32),
    )(p4, W1, b1.reshape(1, d), W2, b2.reshape(1, d))

    sc_fn = _make_sc_gather(B * S, vocab, n_prompt, d, S)
    out = sc_fn(input_ids.reshape(-1), embed_table, enc)
    return out.reshape(B, S, d)


# speedup vs baseline: 2.8655x; 1.3856x over previous
import functools

import jax
import jax.numpy as jnp
from jax import lax
from jax.experimental import pallas as pl
from jax.experimental.pallas import tpu as pltpu
from jax.experimental.pallas import tpu_sc as plsc


def _make_mlp(B, n_prompt, d):
    def body(tids_ref, prompt_ref, task_ref, w1_ref, b1_ref, w2_ref,
             b2_ref, o_ref):
        parts = []
        for b in range(B):
            t = tids_ref[b]
            trow = task_ref[pl.ds(t, 1), :]
            parts.append(prompt_ref[...] + trow)
        p4 = jnp.concatenate(parts, axis=0)
        h = jnp.dot(p4, w1_ref[...],
                    preferred_element_type=jnp.float32) + b1_ref[...]
        h = jnp.maximum(h, 0.0)
        o_ref[...] = jnp.dot(h, w2_ref[...],
                             preferred_element_type=jnp.float32) + b2_ref[...]

    return pl.pallas_call(
        body,
        out_shape=jax.ShapeDtypeStruct((B * n_prompt, d), jnp.float32),
        in_specs=[
            pl.BlockSpec(memory_space=pltpu.SMEM),
            pl.BlockSpec(memory_space=pltpu.VMEM),
            pl.BlockSpec(memory_space=pltpu.VMEM),
            pl.BlockSpec(memory_space=pltpu.VMEM),
            pl.BlockSpec(memory_space=pltpu.VMEM),
            pl.BlockSpec(memory_space=pltpu.VMEM),
            pl.BlockSpec(memory_space=pltpu.VMEM),
        ],
    )


@functools.lru_cache(maxsize=None)
def _make_sc_gather(n_rows, vocab, n_prompt, d, seq_len):
    info = plsc.get_sparse_core_info()
    nc, ns, L = info.num_cores, info.num_subcores, info.num_lanes
    nw = nc * ns
    rpw = n_rows // nw
    CH = 32
    n_ch = rpw // CH
    vec_per_ch = CH // L

    mesh = plsc.VectorSubcoreMesh(core_axis_name="c", subcore_axis_name="s")

    @functools.partial(
        pl.kernel, mesh=mesh,
        out_type=jax.ShapeDtypeStruct((n_rows, d), jnp.float32),
        scratch_types=[
            pltpu.VMEM((rpw,), jnp.int32),
            pltpu.VMEM((rpw,), jnp.int32),
            pltpu.VMEM((CH, d), jnp.float32),
            pltpu.VMEM((CH, d), jnp.float32),
            pltpu.SemaphoreType.DMA,
            pltpu.SemaphoreType.DMA,
            pltpu.SemaphoreType.DMA,
            pltpu.SemaphoreType.DMA,
        ],
    )
    def sc_fn(ids_hbm, table_hbm, enc_hbm, out_hbm, ids_v, cln_v,
              buf0, buf1, sg0, sg1, ss0, ss1):
        bufs = (buf0, buf1)
        sgs = (sg0, sg1)
        sss = (ss0, ss1)
        wid = lax.axis_index("s") * nc + lax.axis_index("c")
        base = wid * rpw
        enc_base = (wid * rpw // seq_len) * n_prompt
        pltpu.sync_copy(ids_hbm.at[pl.ds(base, rpw)], ids_v)

        def build(v, carry):
            off = pl.multiple_of(v * L, L)
            ids16 = ids_v[pl.ds(off, L)]
            cln_v[pl.ds(off, L)] = jnp.where(ids16 >= vocab, 0, ids16)
            return carry
        lax.fori_loop(0, rpw // L, build, 0)

        def gather(g, buf, sem):
            return pltpu.async_copy(
                table_hbm.at[cln_v.at[pl.ds(g * CH, CH)]], buf, sem)

        def gather_wait(g, buf, sem):
            pltpu.make_async_copy(
                table_hbm.at[cln_v.at[pl.ds(g * CH, CH)]], buf, sem).wait()

        def scatter(g, buf, sem):
            return pltpu.async_copy(
                buf, out_hbm.at[pl.ds(base + g * CH, CH)], sem)

        def scatter_wait(g, buf, sem):
            pltpu.make_async_copy(
                buf, out_hbm.at[pl.ds(base + g * CH, CH)], sem).wait()

        def fixup(g, buf):
            def do_vec(vv, carry2):
                off = pl.multiple_of(g * CH + vv * L, L)
                ids16 = ids_v[pl.ds(off, L)]
                for lidx in range(L):
                    idl = ids16[lidx]

                    @pl.when(idl >= vocab)
                    def _(idl=idl, lidx=lidx, vv=vv):
                        erow = enc_base + jnp.minimum(
                            idl - vocab, n_prompt - 1)
                        pltpu.sync_copy(enc_hbm.at[erow],
                                        buf.at[vv * L + lidx])
                return carry2
            lax.fori_loop(0, vec_per_ch, do_vec, 0)

        gather(0, buf0, sg0)

        def run(g2, carry):
            for phase in range(2):
                g = g2 * 2 + phase
                buf, sg, ss = bufs[phase], sgs[phase], sss[phase]
                obuf, osg, oss = (bufs[1 - phase], sgs[1 - phase],
                                  sss[1 - phase])
                gather_wait(g, buf, sg)

                @pl.when(g + 1 < n_ch)
                def _(g=g, obuf=obuf, osg=osg, oss=oss):
                    @pl.when(g >= 1)
                    def _():
                        scatter_wait(g - 1, obuf, oss)
                    gather(g + 1, obuf, osg)

                fixup(g, buf)
                scatter(g, buf, ss)
            return carry
        lax.fori_loop(0, n_ch // 2, run, 0)

        scatter_wait(n_ch - 2, buf0, ss0)
        scatter_wait(n_ch - 1, buf1, ss1)

    return sc_fn


def kernel(input_ids, tids, embed_table, prompt_table, task_table, W1, b1, W2, b2):
    B, S = input_ids.shape
    vocab, d = embed_table.shape
    n_prompt = prompt_table.shape[0]

    enc = _make_mlp(B, n_prompt, d)(
        tids, prompt_table, task_table, W1, b1.reshape(1, d), W2,
        b2.reshape(1, d))

    sc_fn = _make_sc_gather(B * S, vocab, n_prompt, d, S)
    out = sc_fn(input_ids.reshape(-1), embed_table, enc)
    return out.reshape(B, S, d)


# speedup vs baseline: 2.9286x; 1.0220x over previous
import functools

import jax
import jax.numpy as jnp
from jax import lax
from jax.experimental import pallas as pl
from jax.experimental.pallas import tpu as pltpu
from jax.experimental.pallas import tpu_sc as plsc


def _make_mlp(B, n_prompt, d):
    def body(tids_ref, prompt_ref, task_ref, w1_ref, b1_ref, w2_ref,
             b2_ref, o_ref):
        parts = []
        for b in range(B):
            t = tids_ref[b]
            trow = task_ref[pl.ds(t, 1), :]
            parts.append(prompt_ref[...] + trow)
        p4 = jnp.concatenate(parts, axis=0)
        h = jnp.dot(p4, w1_ref[...],
                    preferred_element_type=jnp.float32) + b1_ref[...]
        h = jnp.maximum(h, 0.0)
        o_ref[...] = jnp.dot(h, w2_ref[...],
                             preferred_element_type=jnp.float32) + b2_ref[...]

    return pl.pallas_call(
        body,
        out_shape=jax.ShapeDtypeStruct((B * n_prompt, d), jnp.float32),
        in_specs=[
            pl.BlockSpec(memory_space=pltpu.SMEM),
            pl.BlockSpec(memory_space=pltpu.VMEM),
            pl.BlockSpec(memory_space=pltpu.VMEM),
            pl.BlockSpec(memory_space=pltpu.VMEM),
            pl.BlockSpec(memory_space=pltpu.VMEM),
            pl.BlockSpec(memory_space=pltpu.VMEM),
            pl.BlockSpec(memory_space=pltpu.VMEM),
        ],
    )


@functools.lru_cache(maxsize=None)
def _make_sc_gather(batch, vocab, n_prompt, d, seq_len):
    info = plsc.get_sparse_core_info()
    nc, ns, L = info.num_cores, info.num_subcores, info.num_lanes
    nw = nc * ns
    n_rows = batch * seq_len
    rpw = n_rows // nw
    wpb = seq_len // rpw
    CH = 16
    NBUF = 4
    LOOK = 2
    n_ch = rpw // CH
    vec_per_ch = CH // L

    mesh = plsc.VectorSubcoreMesh(core_axis_name="c", subcore_axis_name="s")

    @functools.partial(
        pl.kernel, mesh=mesh,
        out_type=jax.ShapeDtypeStruct((n_rows, d), jnp.float32),
        scratch_types=[
            pltpu.VMEM((rpw,), jnp.int32),
            pltpu.VMEM((rpw,), jnp.int32),
        ] + [pltpu.VMEM((CH, d), jnp.float32) for _ in range(NBUF)]
          + [pltpu.SemaphoreType.DMA for _ in range(2 * NBUF)],
    )
    def sc_fn(ids_hbm, table_hbm, enc_hbm, out_hbm, ids_v, cln_v, *bufsem):
        bufs = bufsem[:NBUF]
        sgs = bufsem[NBUF:2 * NBUF]
        sss = bufsem[2 * NBUF:]
        wid = lax.axis_index("s") * nc + lax.axis_index("c")
        base = wid * rpw
        bb = wid // wpb
        enc_base = bb * n_prompt
        pltpu.sync_copy(ids_hbm.at[bb, pl.ds((wid % wpb) * rpw, rpw)], ids_v)

        def build(v, carry):
            off = pl.multiple_of(v * L, L)
            ids16 = ids_v[pl.ds(off, L)]
            cln_v[pl.ds(off, L)] = jnp.where(ids16 >= vocab, 0, ids16)
            return carry
        lax.fori_loop(0, rpw // L, build, 0)

        def gather(g, buf, sem):
            return pltpu.async_copy(
                table_hbm.at[cln_v.at[pl.ds(g * CH, CH)]], buf, sem)

        def gather_wait(g, buf, sem):
            pltpu.make_async_copy(
                table_hbm.at[cln_v.at[pl.ds(g * CH, CH)]], buf, sem).wait()

        def scatter(g, buf, sem):
            return pltpu.async_copy(
                buf, out_hbm.at[pl.ds(base + g * CH, CH)], sem)

        def scatter_wait(g, buf, sem):
            pltpu.make_async_copy(
                buf, out_hbm.at[pl.ds(base + g * CH, CH)], sem).wait()

        def fixup(g, buf):
            def do_vec(vv, carry2):
                off = pl.multiple_of(g * CH + vv * L, L)
                ids16 = ids_v[pl.ds(off, L)]
                for lidx in range(L):
                    idl = ids16[lidx]

                    @pl.when(idl >= vocab)
                    def _(idl=idl, lidx=lidx, vv=vv):
                        erow = enc_base + jnp.minimum(
                            idl - vocab, n_prompt - 1)
                        pltpu.sync_copy(enc_hbm.at[erow],
                                        buf.at[vv * L + lidx])
                return carry2
            lax.fori_loop(0, vec_per_ch, do_vec, 0)

        for g in range(LOOK):
            gather(g, bufs[g % NBUF], sgs[g % NBUF])

        def run(gq, carry):
            for phase in range(NBUF):
                g = gq * NBUF + phase
                buf, sg, ss = bufs[phase], sgs[phase], sss[phase]
                nx = (phase + LOOK) % NBUF
                gather_wait(g, buf, sg)

                @pl.when(g + LOOK < n_ch)
                def _(g=g, nx=nx):
                    @pl.when(g + LOOK >= NBUF)
                    def _():
                        scatter_wait(g + LOOK - NBUF, bufs[nx], sss[nx])
                    gather(g + LOOK, bufs[nx], sgs[nx])

                fixup(g, buf)
                scatter(g, buf, ss)
            return carry
        lax.fori_loop(0, n_ch // NBUF, run, 0)

        for g in range(n_ch - NBUF, n_ch):
            scatter_wait(g, bufs[g % NBUF], sss[g % NBUF])

    return sc_fn


def kernel(input_ids, tids, embed_table, prompt_table, task_table, W1, b1, W2, b2):
    B, S = input_ids.shape
    vocab, d = embed_table.shape
    n_prompt = prompt_table.shape[0]

    enc = _make_mlp(B, n_prompt, d)(
        tids, prompt_table, task_table, W1, b1.reshape(1, d), W2,
        b2.reshape(1, d))

    sc_fn = _make_sc_gather(B, vocab, n_prompt, d, S)
    out = sc_fn(input_ids, embed_table, enc)
    return out.reshape(B, S, d)
